# baseline (speedup 1.0000x reference)
import functools

import jax
import jax.numpy as jnp
from jax import lax
from jax.experimental import pallas as pl
from jax.experimental.pallas import tpu as pltpu
from jax.experimental.pallas import tpu_sc as plsc

NC, NS = 2, 16
N = 10000
E = 160000
CH = 256
H = CH // 2

CHUNK = 80
EP = 163840
NPAD = 10240
TRASH = N
DW = 16

ROWS_PER_TILE = NPAD // NS
DEG_CHUNKS = EP // (NC * NS * CHUNK)
GS_CHUNKS = EP // (NS * CHUNK)
NBUF = 2
IBLK = 16
NIB = GS_CHUNKS // IBLK

_mesh = plsc.VectorSubcoreMesh(
    core_axis_name="c", subcore_axis_name="s", num_cores=NC, num_subcores=NS)


@functools.partial(
    pl.kernel,
    out_type=jax.ShapeDtypeStruct((NC, NPAD, H), jnp.float32),
    mesh=_mesh,
    scratch_types=[
        pltpu.VMEM_SHARED((NPAD, H), jnp.float32),
        pltpu.VMEM((DEG_CHUNKS, CHUNK), jnp.int32),
        pltpu.VMEM((CHUNK, H), jnp.float32),
        pltpu.SemaphoreType.DMA,
    ],
)
def _deg_kernel(idx_hbm, out_hbm, acc, idxv, ones_v, sem):
    c = lax.axis_index("c")
    s = lax.axis_index("s")
    w = c * NS + s

    @pl.loop(0, CHUNK, unroll=8)
    def _fill(i):
        @pl.loop(0, H // 16, unroll=8)
        def _fill2(j):
            ones_v[i, pl.ds(j * 16, 16)] = jnp.zeros((16,), jnp.float32)

    @pl.loop(0, ROWS_PER_TILE // CHUNK)
    def _zero(k):
        pltpu.sync_copy(ones_v, acc.at[pl.ds(s * ROWS_PER_TILE + k * CHUNK, CHUNK)])

    @pl.loop(0, CHUNK, unroll=8)
    def _fill1(i):
        @pl.loop(0, H // 16, unroll=8)
        def _fill12(j):
            ones_v[i, pl.ds(j * 16, 16)] = jnp.ones((16,), jnp.float32)

    plsc.subcore_barrier()

    pltpu.sync_copy(idx_hbm.at[pl.ds(w * DEG_CHUNKS, DEG_CHUNKS)], idxv)

    @pl.loop(0, DEG_CHUNKS)
    def _hist(j):
        pltpu.async_copy(ones_v, acc.at[idxv.at[j]], sem, add=True)

    @pl.loop(0, DEG_CHUNKS)
    def _drain(j):
        pltpu.make_async_copy(ones_v, acc.at[idxv.at[j]], sem).wait()

    plsc.subcore_barrier()
    pltpu.sync_copy(
        acc.at[pl.ds(s * ROWS_PER_TILE, ROWS_PER_TILE)],
        out_hbm.at[c, pl.ds(s * ROWS_PER_TILE, ROWS_PER_TILE)],
    )


@functools.partial(
    pl.kernel,
    out_type=jax.ShapeDtypeStruct((NC, NPAD, H), jnp.float32),
    mesh=_mesh,
    scratch_types=[
        pltpu.VMEM_SHARED((NPAD, H), jnp.float32),
        pltpu.VMEM((2, IBLK, CHUNK), jnp.int32),
        pltpu.VMEM((2, IBLK, CHUNK), jnp.int32),
        pltpu.VMEM((NBUF, CHUNK, H), jnp.float32),
        [pltpu.SemaphoreType.DMA] * NBUF,
        [pltpu.SemaphoreType.DMA] * NBUF,
        pltpu.SemaphoreType.DMA,
    ],
)
def _gs_kernel(g_hbm, ridx_hbm, cidx_hbm, s_hbm, acc, ridxv, cidxv, bufs, gsems, ssems, isem):
    c = lax.axis_index("c")
    s = lax.axis_index("s")

    @pl.loop(0, CHUNK, unroll=8)
    def _zb(i):
        @pl.loop(0, H // 16, unroll=8)
        def _zb2(j):
            bufs[0, i, pl.ds(j * 16, 16)] = jnp.zeros((16,), jnp.float32)

    @pl.loop(0, ROWS_PER_TILE // CHUNK)
    def _zero(k):
        pltpu.sync_copy(bufs.at[0], acc.at[pl.ds(s * ROWS_PER_TILE + k * CHUNK, CHUNK)])

    plsc.subcore_barrier()

    def _idx_copies(ib, sl):
        base = s * GS_CHUNKS + ib * IBLK
        return (
            pltpu.make_async_copy(ridx_hbm.at[c, pl.ds(base, IBLK)], ridxv.at[sl], isem),
            pltpu.make_async_copy(cidx_hbm.at[pl.ds(base, IBLK)], cidxv.at[sl], isem),
        )

    for cp in _idx_copies(0, 0):
        cp.start()

    for ib in range(NIB):
        sl = ib % 2
        for cp in _idx_copies(ib, sl):
            cp.wait()
        if ib + 1 < NIB:
            for cp in _idx_copies(ib + 1, 1 - sl):
                cp.start()

        for b in range(NBUF):
            pltpu.async_copy(g_hbm.at[ridxv.at[sl, b]], bufs.at[b], gsems[b])

        @pl.loop(0, IBLK // NBUF)
        def _pipe(q):
            for b in range(NBUF):
                r = q * NBUF + b
                pltpu.make_async_copy(g_hbm.at[ridxv.at[sl, r]], bufs.at[b], gsems[b]).wait()
                pltpu.async_copy(bufs.at[b], acc.at[cidxv.at[sl, r]], ssems[b], add=True)
            for b in range(NBUF):
                r = q * NBUF + b
                pltpu.make_async_copy(bufs.at[b], acc.at[cidxv.at[sl, r]], ssems[b]).wait()

                @pl.when(q < IBLK // NBUF - 1)
                def _prefetch():
                    pltpu.async_copy(g_hbm.at[ridxv.at[sl, r + NBUF]], bufs.at[b], gsems[b])

    plsc.subcore_barrier()
    pltpu.sync_copy(
        acc.at[pl.ds(s * ROWS_PER_TILE, ROWS_PER_TILE)],
        s_hbm.at[c, pl.ds(s * ROWS_PER_TILE, ROWS_PER_TILE)],
    )


_BLK = 400
_GRID = N // _BLK


def _tc1_body(x_ref, w_ref, b_ref, d_ref, g_ref, base_ref):
    h = jnp.dot(x_ref[...], w_ref[...], preferred_element_type=jnp.float32)
    deg = d_ref[0] + d_ref[1] + 1.0
    dis = lax.rsqrt(deg)
    base_ref[...] = h * (1.0 / deg) + b_ref[...]
    g_ref[0] = h[:, :H] * dis
    g_ref[1] = h[:, H:] * dis


def _tc2_body(s_ref, d_ref, base_ref, o_ref):
    deg = d_ref[0] + d_ref[1] + 1.0
    dis = lax.rsqrt(deg)
    o_ref[:, :H] = s_ref[0] * dis + base_ref[:, :H]
    o_ref[:, H:] = s_ref[1] * dis + base_ref[:, H:]


def kernel(x, edge_index, weight, bias):
    row = edge_index[0]
    col = edge_index[1]
    pad = EP - E

    rowp = jnp.concatenate([row, jnp.zeros((pad,), jnp.int32)])
    colp = jnp.concatenate([col, jnp.full((pad,), TRASH, jnp.int32)])
    rowdeg = jnp.concatenate([row, jnp.full((pad,), TRASH, jnp.int32)])
    row2 = jnp.stack([rowp, rowp + N]).reshape(NC, EP // CHUNK, CHUNK)
    col2 = colp.reshape(EP // CHUNK, CHUNK)
    rowdeg2 = rowdeg.reshape(EP // CHUNK, CHUNK)

    degp = _deg_kernel(rowdeg2)[:, :, 0:1]

    g, base = pl.pallas_call(
        _tc1_body,
        grid=(_GRID,),
        in_specs=[
            pl.BlockSpec((_BLK, CH), lambda i: (i, 0)),
            pl.BlockSpec((CH, CH), lambda i: (0, 0)),
            pl.BlockSpec((1, CH), lambda i: (0, 0)),
            pl.BlockSpec((NC, _BLK, 1), lambda i: (0, i, 0)),
        ],
        out_specs=[
            pl.BlockSpec((NC, _BLK, H), lambda i: (0, i, 0)),
            pl.BlockSpec((_BLK, CH), lambda i: (i, 0)),
        ],
        out_shape=[
            jax.ShapeDtypeStruct((NC, N, H), jnp.float32),
            jax.ShapeDtypeStruct((N, CH), jnp.float32),
        ],
    )(x, weight, bias.reshape(1, CH), degp)

    s_agg = _gs_kernel(g.reshape(NC * N, H), row2, col2)

    out = pl.pallas_call(
        _tc2_body,
        grid=(_GRID,),
        in_specs=[
            pl.BlockSpec((NC, _BLK, H), lambda i: (0, i, 0)),
            pl.BlockSpec((NC, _BLK, 1), lambda i: (0, i, 0)),
            pl.BlockSpec((_BLK, CH), lambda i: (i, 0)),
        ],
        out_specs=pl.BlockSpec((_BLK, CH), lambda i: (i, 0)),
        out_shape=jax.ShapeDtypeStruct((N, CH), jnp.float32),
    )(s_agg, degp, base)

    return out


# speedup vs baseline: 1.1334x; 1.1334x over previous
import functools

import jax
import jax.numpy as jnp
from jax import lax
from jax.experimental import pallas as pl
from jax.experimental.pallas import tpu as pltpu
from jax.experimental.pallas import tpu_sc as plsc

NC, NS = 2, 16
N = 10000
E = 160000
CH = 256
H = CH // 2

CHUNK = 80
EP = 163840
NPAD = 10240
TRASH = N
DW = 16

ROWS_PER_TILE = NPAD // NS
DEG_CHUNKS = EP // (NC * NS * CHUNK)
GS_CHUNKS = EP // (NS * CHUNK)
NBUF = 2
IBLK = 16
NIB = GS_CHUNKS // IBLK

_mesh = plsc.VectorSubcoreMesh(
    core_axis_name="c", subcore_axis_name="s", num_cores=NC, num_subcores=NS)


@functools.partial(
    pl.kernel,
    out_type=jax.ShapeDtypeStruct((NC, NPAD, H), jnp.float32),
    mesh=_mesh,
    scratch_types=[
        pltpu.VMEM_SHARED((NPAD, H), jnp.float32),
        pltpu.VMEM((DEG_CHUNKS, CHUNK), jnp.int32),
        pltpu.VMEM((CHUNK, H), jnp.float32),
        pltpu.SemaphoreType.DMA,
    ],
)
def _deg_kernel(idx_hbm, out_hbm, acc, idxv, ones_v, sem):
    c = lax.axis_index("c")
    s = lax.axis_index("s")
    w = c * NS + s

    @pl.loop(0, CHUNK, unroll=8)
    def _fill(i):
        @pl.loop(0, H // 16, unroll=8)
        def _fill2(j):
            ones_v[i, pl.ds(j * 16, 16)] = jnp.zeros((16,), jnp.float32)

    @pl.loop(0, ROWS_PER_TILE // CHUNK)
    def _zero(k):
        pltpu.sync_copy(ones_v, acc.at[pl.ds(s * ROWS_PER_TILE + k * CHUNK, CHUNK)])

    @pl.loop(0, CHUNK, unroll=8)
    def _fill1(i):
        @pl.loop(0, H // 16, unroll=8)
        def _fill12(j):
            ones_v[i, pl.ds(j * 16, 16)] = jnp.ones((16,), jnp.float32)

    plsc.subcore_barrier()

    pltpu.sync_copy(idx_hbm.at[pl.ds(w * DEG_CHUNKS, DEG_CHUNKS)], idxv)

    @pl.loop(0, DEG_CHUNKS)
    def _hist(j):
        pltpu.async_copy(ones_v, acc.at[idxv.at[j]], sem, add=True)

    @pl.loop(0, DEG_CHUNKS)
    def _drain(j):
        pltpu.make_async_copy(ones_v, acc.at[idxv.at[j]], sem).wait()

    plsc.subcore_barrier()
    pltpu.sync_copy(
        acc.at[pl.ds(s * ROWS_PER_TILE, ROWS_PER_TILE)],
        out_hbm.at[c, pl.ds(s * ROWS_PER_TILE, ROWS_PER_TILE)],
    )


@functools.partial(
    pl.kernel,
    out_type=jax.ShapeDtypeStruct((NC, NPAD, H), jnp.float32),
    mesh=_mesh,
    scratch_types=[
        pltpu.VMEM_SHARED((NPAD, H), jnp.float32),
        pltpu.VMEM((2, IBLK, CHUNK), jnp.int32),
        pltpu.VMEM((2, IBLK, CHUNK), jnp.int32),
        pltpu.VMEM((NBUF, CHUNK, H), jnp.float32),
        [pltpu.SemaphoreType.DMA] * NBUF,
        [pltpu.SemaphoreType.DMA] * NBUF,
        pltpu.SemaphoreType.DMA,
    ],
)
def _gs_kernel(g_hbm, ridx_hbm, cidx_hbm, s_hbm, acc, ridxv, cidxv, bufs, gsems, ssems, isem):
    c = lax.axis_index("c")
    s = lax.axis_index("s")

    @pl.loop(0, CHUNK, unroll=8)
    def _zb(i):
        @pl.loop(0, H // 16, unroll=8)
        def _zb2(j):
            bufs[0, i, pl.ds(j * 16, 16)] = jnp.zeros((16,), jnp.float32)

    @pl.loop(0, ROWS_PER_TILE // CHUNK)
    def _zero(k):
        pltpu.sync_copy(bufs.at[0], acc.at[pl.ds(s * ROWS_PER_TILE + k * CHUNK, CHUNK)])

    plsc.subcore_barrier()

    def _idx_copies(ib, sl):
        base = s * GS_CHUNKS + ib * IBLK
        return (
            pltpu.make_async_copy(ridx_hbm.at[c, pl.ds(base, IBLK)], ridxv.at[sl], isem),
            pltpu.make_async_copy(cidx_hbm.at[pl.ds(base, IBLK)], cidxv.at[sl], isem),
        )

    for cp in _idx_copies(0, 0):
        cp.start()

    for ib in range(NIB):
        sl = ib % 2
        for cp in _idx_copies(ib, sl):
            cp.wait()
        if ib + 1 < NIB:
            for cp in _idx_copies(ib + 1, 1 - sl):
                cp.start()

        for b in range(NBUF):
            pltpu.async_copy(g_hbm.at[ridxv.at[sl, b]], bufs.at[b], gsems[b])

        @pl.loop(0, IBLK // NBUF)
        def _pipe(q):
            for b in range(NBUF):
                r = q * NBUF + b
                pltpu.make_async_copy(g_hbm.at[ridxv.at[sl, r]], bufs.at[b], gsems[b]).wait()
                pltpu.async_copy(bufs.at[b], acc.at[cidxv.at[sl, r]], ssems[b], add=True)
            for b in range(NBUF):
                r = q * NBUF + b
                pltpu.make_async_copy(bufs.at[b], acc.at[cidxv.at[sl, r]], ssems[b]).wait()

                @pl.when(q < IBLK // NBUF - 1)
                def _prefetch():
                    pltpu.async_copy(g_hbm.at[ridxv.at[sl, r + NBUF]], bufs.at[b], gsems[b])

    plsc.subcore_barrier()
    pltpu.sync_copy(
        acc.at[pl.ds(s * ROWS_PER_TILE, ROWS_PER_TILE)],
        s_hbm.at[c, pl.ds(s * ROWS_PER_TILE, ROWS_PER_TILE)],
    )


_BLK = 400
_GRID = N // _BLK


def _mm_body(x_ref, w_ref, h_ref):
    h_ref[...] = jnp.dot(x_ref[...], w_ref[...], preferred_element_type=jnp.float32)


def _tc1_body(h_ref, b_ref, d_ref, g_ref, base_ref):
    h = h_ref[...]
    deg = d_ref[0] + d_ref[1] + 1.0
    dis = lax.rsqrt(deg)
    base_ref[...] = h * (1.0 / deg) + b_ref[...]
    g_ref[0] = h[:, :H] * dis
    g_ref[1] = h[:, H:] * dis


def _tc2_body(s_ref, d_ref, base_ref, o_ref):
    deg = d_ref[0] + d_ref[1] + 1.0
    dis = lax.rsqrt(deg)
    o_ref[:, :H] = s_ref[0] * dis + base_ref[:, :H]
    o_ref[:, H:] = s_ref[1] * dis + base_ref[:, H:]


def kernel(x, edge_index, weight, bias):
    row = edge_index[0]
    col = edge_index[1]
    pad = EP - E

    rowp = jnp.concatenate([row, jnp.zeros((pad,), jnp.int32)])
    colp = jnp.concatenate([col, jnp.full((pad,), TRASH, jnp.int32)])
    rowdeg = jnp.concatenate([row, jnp.full((pad,), TRASH, jnp.int32)])
    row2 = jnp.stack([rowp, rowp + N]).reshape(NC, EP // CHUNK, CHUNK)
    col2 = colp.reshape(EP // CHUNK, CHUNK)
    rowdeg2 = rowdeg.reshape(EP // CHUNK, CHUNK)

    degp = _deg_kernel(rowdeg2)[:, :, 0:1]

    h = pl.pallas_call(
        _mm_body,
        grid=(_GRID,),
        in_specs=[
            pl.BlockSpec((_BLK, CH), lambda i: (i, 0)),
            pl.BlockSpec((CH, CH), lambda i: (0, 0)),
        ],
        out_specs=pl.BlockSpec((_BLK, CH), lambda i: (i, 0)),
        out_shape=jax.ShapeDtypeStruct((N, CH), jnp.float32),
    )(x, weight)

    g, base = pl.pallas_call(
        _tc1_body,
        grid=(_GRID,),
        in_specs=[
            pl.BlockSpec((_BLK, CH), lambda i: (i, 0)),
            pl.BlockSpec((1, CH), lambda i: (0, 0)),
            pl.BlockSpec((NC, _BLK, 1), lambda i: (0, i, 0)),
        ],
        out_specs=[
            pl.BlockSpec((NC, _BLK, H), lambda i: (0, i, 0)),
            pl.BlockSpec((_BLK, CH), lambda i: (i, 0)),
        ],
        out_shape=[
            jax.ShapeDtypeStruct((NC, N, H), jnp.float32),
            jax.ShapeDtypeStruct((N, CH), jnp.float32),
        ],
    )(h, bias.reshape(1, CH), degp)

    s_agg = _gs_kernel(g.reshape(NC * N, H), row2, col2)

    out = pl.pallas_call(
        _tc2_body,
        grid=(_GRID,),
        in_specs=[
            pl.BlockSpec((NC, _BLK, H), lambda i: (0, i, 0)),
            pl.BlockSpec((NC, _BLK, 1), lambda i: (0, i, 0)),
            pl.BlockSpec((_BLK, CH), lambda i: (i, 0)),
        ],
        out_specs=pl.BlockSpec((_BLK, CH), lambda i: (i, 0)),
        out_shape=jax.ShapeDtypeStruct((N, CH), jnp.float32),
    )(s_agg, degp, base)

    return out
